# baseline (speedup 1.0000x reference)
import functools

import jax
import jax.numpy as jnp
from jax.experimental import pallas as pl
from jax.experimental.pallas import tpu as pltpu

_NUM_EXPERTS = 64
_TOPK = 8
_TEMP = 1.0
_HIDDEN = 4096
_BM = 1024


def _route(gate, gb):
    work = gate + gb
    row = jax.lax.broadcasted_iota(jnp.int32, gate.shape, 0)
    row_rev = jnp.int32(_NUM_EXPERTS - 1) - row

    bits = jax.lax.bitcast_convert_type(work, jnp.int32)
    skey = bits ^ ((bits >> 31) & jnp.int32(0x7FFFFFFF))

    sentinel = jnp.int32(-(2 ** 31))
    ids_rows = []
    for _ in range(_TOPK):
        m = jnp.max(skey, axis=0, keepdims=True)
        eq = skey == m
        sel = jnp.max(jnp.where(eq, row_rev, jnp.int32(-1)),
                      axis=0, keepdims=True)
        ids_rows.append(jnp.int32(_NUM_EXPERTS - 1) - sel)
        skey = jnp.where(eq & (row_rev == sel), sentinel, skey)
    ids = jnp.concatenate(ids_rows, axis=0)

    selected = skey == sentinel
    e = jnp.where(selected, jnp.exp(gate), 0.0)
    return e / jnp.sum(e, axis=0, keepdims=True), ids


def _router_block(x_ref, w_ref, b_ref, gb_ref, out_ref, ids_ref):
    w = w_ref[...]
    b = b_ref[...]
    gb = gb_ref[...]
    half = _BM // 2
    gates = []
    for h in range(2):
        xh = x_ref[pl.ds(h * half, half), :]
        g = jax.lax.dot_general(
            w, xh, (((1,), (1,)), ((), ())),
            preferred_element_type=jnp.float32,
        )
        gates.append(g * (1.0 / _TEMP) + b)
    for h in range(2):
        out_h, ids_h = _route(gates[h], gb)
        out_ref[:, pl.ds(h * half, half)] = out_h
        ids_ref[:, pl.ds(h * half, half)] = ids_h


@functools.partial(jax.jit, static_argnames=())
def kernel(x, W, b, gate_bias):
    B, S, H = x.shape
    M = B * S
    x2 = x.reshape(M, H)
    b2 = b.reshape(_NUM_EXPERTS, 1)
    gb2 = gate_bias.reshape(_NUM_EXPERTS, 1)

    grid = (M // _BM,)
    out_t, ids_t = pl.pallas_call(
        _router_block,
        grid=grid,
        in_specs=[
            pl.BlockSpec((_BM, H), lambda i: (i, 0)),
            pl.BlockSpec((_NUM_EXPERTS, H), lambda i: (0, 0)),
            pl.BlockSpec((_NUM_EXPERTS, 1), lambda i: (0, 0)),
            pl.BlockSpec((_NUM_EXPERTS, 1), lambda i: (0, 0)),
        ],
        out_specs=[
            pl.BlockSpec((_NUM_EXPERTS, _BM), lambda i: (0, i)),
            pl.BlockSpec((_TOPK, _BM), lambda i: (0, i)),
        ],
        out_shape=[
            jax.ShapeDtypeStruct((_NUM_EXPERTS, M), jnp.float32),
            jax.ShapeDtypeStruct((_TOPK, M), jnp.int32),
        ],
        compiler_params=pltpu.CompilerParams(
            dimension_semantics=("arbitrary",),
        ),
    )(x2, W, b2, gb2)
    out = out_t.T.reshape(B, S, _NUM_EXPERTS)
    ids = ids_t.T.reshape(B, S, _TOPK)
    return out, ids


# speedup vs baseline: 1.0219x; 1.0219x over previous
import functools

import jax
import jax.numpy as jnp
from jax.experimental import pallas as pl
from jax.experimental.pallas import tpu as pltpu

_NUM_EXPERTS = 64
_TOPK = 8
_TEMP = 1.0
_HIDDEN = 4096
_BM = 1024


def _route(gate, gb):
    work = gate + gb
    row = jax.lax.broadcasted_iota(jnp.int32, gate.shape, 0)
    row_rev_f = (jnp.int32(_NUM_EXPERTS - 1) - row).astype(jnp.float32)

    neg_inf = jnp.float32(-jnp.inf)
    ids_rows = []
    for _ in range(_TOPK):
        m = jnp.max(work, axis=0, keepdims=True)
        sel = jnp.max(jnp.where(work == m, row_rev_f, jnp.float32(-1.0)),
                      axis=0, keepdims=True)
        ids_rows.append(
            (jnp.float32(_NUM_EXPERTS - 1) - sel).astype(jnp.int32))
        work = jnp.where(row_rev_f == sel, neg_inf, work)
    ids = jnp.concatenate(ids_rows, axis=0)

    selected = work == neg_inf
    e = jnp.where(selected, jnp.exp(gate), 0.0)
    return e / jnp.sum(e, axis=0, keepdims=True), ids


def _router_block(x_ref, w_ref, b_ref, gb_ref, out_ref, ids_ref):
    w = w_ref[...]
    b = b_ref[...]
    gb = gb_ref[...]
    half = _BM // 2
    gates = []
    for h in range(2):
        xh = x_ref[pl.ds(h * half, half), :]
        g = jax.lax.dot_general(
            w, xh, (((1,), (1,)), ((), ())),
            preferred_element_type=jnp.float32,
        )
        gates.append(g * (1.0 / _TEMP) + b)
    for h in range(2):
        out_h, ids_h = _route(gates[h], gb)
        out_ref[:, pl.ds(h * half, half)] = out_h
        ids_ref[:, pl.ds(h * half, half)] = ids_h


@functools.partial(jax.jit, static_argnames=())
def kernel(x, W, b, gate_bias):
    B, S, H = x.shape
    M = B * S
    x2 = x.reshape(M, H)
    b2 = b.reshape(_NUM_EXPERTS, 1)
    gb2 = gate_bias.reshape(_NUM_EXPERTS, 1)

    grid = (M // _BM,)
    out_t, ids_t = pl.pallas_call(
        _router_block,
        grid=grid,
        in_specs=[
            pl.BlockSpec((_BM, H), lambda i: (i, 0)),
            pl.BlockSpec((_NUM_EXPERTS, H), lambda i: (0, 0)),
            pl.BlockSpec((_NUM_EXPERTS, 1), lambda i: (0, 0)),
            pl.BlockSpec((_NUM_EXPERTS, 1), lambda i: (0, 0)),
        ],
        out_specs=[
            pl.BlockSpec((_NUM_EXPERTS, _BM), lambda i: (0, i)),
            pl.BlockSpec((_TOPK, _BM), lambda i: (0, i)),
        ],
        out_shape=[
            jax.ShapeDtypeStruct((_NUM_EXPERTS, M), jnp.float32),
            jax.ShapeDtypeStruct((_TOPK, M), jnp.int32),
        ],
        compiler_params=pltpu.CompilerParams(
            dimension_semantics=("arbitrary",),
        ),
    )(x2, W, b2, gb2)
    out = out_t.T.reshape(B, S, _NUM_EXPERTS)
    ids = ids_t.T.reshape(B, S, _TOPK)
    return out, ids
